# baseline (speedup 1.0000x reference)
import jax
import jax.numpy as jnp
from jax.experimental import pallas as pl
from jax.experimental.pallas import tpu as pltpu

_K = 65536
_GRID = 4
_B = _K // _GRID
_AB = _K // 128 // _GRID

_LN2 = 0.6931471805599453


def _loss_kernel(attx_ref, attg_ref, spap_ref, spat_ref, conp_ref, cont_ref,
                 out_ref, ce_acc, spa_acc, con_acc):
    i = pl.program_id(0)

    @pl.when(i == 0)
    def _init():
        ce_acc[...] = jnp.zeros_like(ce_acc)
        spa_acc[...] = jnp.zeros_like(spa_acc)
        con_acc[...] = jnp.zeros_like(con_acc)

    x0 = attx_ref[0]
    x1 = attx_ref[1]
    x2 = attx_ref[2]
    g = attg_ref[...]
    m = jnp.maximum(jnp.maximum(x0, x1), x2)
    s = jnp.exp(x0 - m) + jnp.exp(x1 - m) + jnp.exp(x2 - m)
    lse = m + jnp.log(s)
    xl = jnp.where(g == 0, x0, jnp.where(g == 1, x1, x2))
    ce_acc[...] += lse - xl

    ps = spap_ref[...]
    qs = jnp.where(spat_ref[...] == 1, ps, 1.0 - ps)
    spa_acc[...] += jnp.log2(jnp.maximum(qs, 1e-7))

    pc = conp_ref[...]
    qc = jnp.where(cont_ref[...] == 1, pc, 1.0 - pc)
    lc = jnp.log2(jnp.maximum(qc, 1e-7))
    con_acc[...] += jnp.sum(lc, axis=0, keepdims=True)

    @pl.when(i == _GRID - 1)
    def _fin():
        att = jnp.sum(ce_acc[...]) * (1.0 / _K)
        spa = jnp.sum(spa_acc[...]) * (-_LN2 / (_K * 6))
        con = jnp.sum(con_acc[...]) * (-_LN2 / (_K * 17))
        out_ref[0] = att
        out_ref[1] = spa
        out_ref[2] = con
        out_ref[3] = att + spa + con


def kernel(att_logits, spa_probs, con_probs, att_gt, spa_gt, con_gt):
    attx = att_logits.T.reshape(3, _K // 128, 128)
    attg = att_gt.astype(jnp.int32).reshape(_K // 128, 128)
    spap = spa_probs.T
    spat = spa_gt.T
    conp = con_probs.T
    cont = con_gt.T

    out = pl.pallas_call(
        _loss_kernel,
        grid=(_GRID,),
        in_specs=[
            pl.BlockSpec((3, _AB, 128), lambda i: (0, i, 0)),
            pl.BlockSpec((_AB, 128), lambda i: (i, 0)),
            pl.BlockSpec((6, _B), lambda i: (0, i)),
            pl.BlockSpec((6, _B), lambda i: (0, i)),
            pl.BlockSpec((17, _B), lambda i: (0, i)),
            pl.BlockSpec((17, _B), lambda i: (0, i)),
        ],
        out_specs=pl.BlockSpec(memory_space=pltpu.MemorySpace.SMEM),
        out_shape=jax.ShapeDtypeStruct((4,), jnp.float32),
        scratch_shapes=[
            pltpu.VMEM((_AB, 128), jnp.float32),
            pltpu.VMEM((6, _B), jnp.float32),
            pltpu.VMEM((1, _B), jnp.float32),
        ],
        compiler_params=pltpu.CompilerParams(
            dimension_semantics=("arbitrary",),
        ),
    )(attx, attg, spap, spat, conp, cont)

    return (out[0], out[1], out[2], out[3])
